# baseline (speedup 1.0000x reference)
import functools

import jax
import jax.numpy as jnp
from jax import lax
from jax.experimental import pallas as pl
from jax.experimental.pallas import tpu as pltpu
from jax.experimental.pallas import tpu_sc as plsc

N = 10000
E = 160000
LATENT = 128
NUM_MP = 10

CHUNK = 128
NCHUNKS = E // CHUNK
NC, NS = 2, 16
NW = NC * NS
STRIPE = 640
STRIPE_LAST = N - 15 * STRIPE

_mesh = plsc.VectorSubcoreMesh(core_axis_name="c", subcore_axis_name="s")


LATW = LATENT


def _make_gather(c_lo, nch):
    eh = nch * CHUNK
    wch = nch // NW
    nrem = nch - wch * NW

    @functools.partial(
        pl.kernel,
        out_type=(jax.ShapeDtypeStruct((eh, LATENT), jnp.float32),
                  jax.ShapeDtypeStruct((eh, LATENT), jnp.float32)),
        mesh=_mesh,
        scratch_types=[
            pltpu.VMEM((wch + 1, 2, CHUNK), jnp.int32),
            pltpu.VMEM((CHUNK, LATENT), jnp.float32),
            pltpu.VMEM((CHUNK, LATENT), jnp.float32),
            pltpu.VMEM((CHUNK, LATENT), jnp.float32),
            pltpu.VMEM((CHUNK, LATENT), jnp.float32),
            pltpu.VMEM((CHUNK, LATENT), jnp.float32),
            pltpu.VMEM((CHUNK, LATENT), jnp.float32),
            pltpu.SemaphoreType.DMA,
            pltpu.SemaphoreType.DMA,
            pltpu.SemaphoreType.DMA,
            pltpu.SemaphoreType.DMA,
            pltpu.SemaphoreType.DMA,
            pltpu.SemaphoreType.DMA,
        ],
    )
    def gather(pq_hbm, sd_hbm, g1_hbm, g2_hbm, slab,
               rp0, rp1, rp2, rq0, rq1, rq2, sg0, sg1, sg2, so0, so1, so2):
        wid = lax.axis_index("s") * NC + lax.axis_index("c")
        c0 = wid * wch
        rp = (rp0, rp1, rp2)
        rq = (rq0, rq1, rq2)
        sg = (sg0, sg1, sg2)
        so = (so0, so1, so2)

        pltpu.sync_copy(sd_hbm.at[pl.ds(c_lo + c0, wch)],
                        slab.at[pl.ds(0, wch)])

        def g_start(j, b):
            pltpu.async_copy(pq_hbm.at[slab.at[j, 0]], rp[b], sg[b])
            pltpu.async_copy(pq_hbm.at[slab.at[j, 1]], rq[b], sg[b])

        def g_wait(j, b):
            pltpu.make_async_copy(pq_hbm.at[slab.at[j, 0]], rp[b], sg[b]).wait()
            pltpu.make_async_copy(pq_hbm.at[slab.at[j, 1]], rq[b], sg[b]).wait()

        def s_start(j, b):
            base = (c0 + j) * CHUNK
            pltpu.async_copy(rp[b], g1_hbm.at[pl.ds(base, CHUNK)], so[b])
            pltpu.async_copy(rq[b], g2_hbm.at[pl.ds(base, CHUNK)], so[b])

        def s_wait(j, b):
            base = (c0 + j) * CHUNK
            pltpu.make_async_copy(rp[b], g1_hbm.at[pl.ds(base, CHUNK)], so[b]).wait()
            pltpu.make_async_copy(rq[b], g2_hbm.at[pl.ds(base, CHUNK)], so[b]).wait()

        g_start(0, 0)
        g_start(1, 1)

        def body(j, carry):
            def step(b):
                g_wait(j, b)
                s_start(j, b)
                prev = (b + 2) % 3

                @pl.when(j >= 1)
                def _():
                    s_wait(j - 1, prev)

                @pl.when(j + 2 < wch)
                def _():
                    g_start(j + 2, prev)

            for k in range(3):
                @pl.when(lax.rem(j, 3) == k)
                def _(k=k):
                    step(k)

            return carry

        lax.fori_loop(0, wch, body, 0)
        s_wait(wch - 1, (wch - 1) % 3)

        @pl.when(wid < nrem)
        def _():
            cr = wch * NW + wid
            pltpu.sync_copy(sd_hbm.at[c_lo + cr], slab.at[wch])
            pltpu.async_copy(pq_hbm.at[slab.at[wch, 0]], rp0, sg0)
            pltpu.async_copy(pq_hbm.at[slab.at[wch, 1]], rq0, sg0)
            pltpu.make_async_copy(pq_hbm.at[slab.at[wch, 0]], rp0, sg0).wait()
            pltpu.make_async_copy(pq_hbm.at[slab.at[wch, 1]], rq0, sg0).wait()
            pltpu.sync_copy(rp0, g1_hbm.at[pl.ds(cr * CHUNK, CHUNK)])
            pltpu.sync_copy(rq0, g2_hbm.at[pl.ds(cr * CHUNK, CHUNK)])

    return gather


NCH_H = NCHUNKS // 2
EH = NCH_H * CHUNK
_sc_gather_a = _make_gather(0, NCH_H)
_sc_gather_b = _make_gather(NCH_H, NCH_H)


WCHS = NCH_H // (NW // 2)
NREMS = NCH_H - WCHS * (NW // 2)


@functools.partial(
    pl.kernel,
    out_type=jax.ShapeDtypeStruct((NC, N, LATENT), jnp.float32),
    mesh=_mesh,
    scratch_types=[
        pltpu.VMEM((WCHS + 1, 1, CHUNK), jnp.int32),
        pltpu.VMEM((CHUNK, LATENT), jnp.float32),
        pltpu.VMEM((CHUNK, LATENT), jnp.float32),
        pltpu.VMEM_SHARED((N, LATENT), jnp.float32),
        pltpu.SemaphoreType.DMA,
        pltpu.SemaphoreType.DMA,
        pltpu.SemaphoreType.DMA,
        pltpu.SemaphoreType.DMA,
    ],
)
def _sc_scatter(ma_hbm, mb_hbm, dst2_hbm, zeros_hbm, out_hbm,
                slab, rm0, rm1, acc_sh, sl0, sl1, ss0, ss1):
    cid = lax.axis_index("c")
    sid = lax.axis_index("s")
    wid = sid * NC + cid
    r0 = sid * STRIPE
    rm = (rm0, rm1)
    sl = (sl0, sl1)
    ss = (ss0, ss1)

    @pl.when(sid < NS - 1)
    def _():
        pltpu.sync_copy(zeros_hbm.at[pl.ds(r0, STRIPE)],
                        acc_sh.at[pl.ds(r0, STRIPE)])

    @pl.when(sid == NS - 1)
    def _():
        pltpu.sync_copy(zeros_hbm.at[pl.ds(r0, STRIPE_LAST)],
                        acc_sh.at[pl.ds(r0, STRIPE_LAST)])

    plsc.subcore_barrier()

    def run_half(m_hbm, c_half, wid_local):
        c0 = wid_local * WCHS

        pltpu.sync_copy(dst2_hbm.at[pl.ds(c_half + c0, WCHS)],
                        slab.at[pl.ds(0, WCHS)])

        def l_start(j, b):
            base = (c0 + j) * CHUNK
            pltpu.async_copy(m_hbm.at[pl.ds(base, CHUNK)], rm[b], sl[b])

        def l_wait(j, b):
            base = (c0 + j) * CHUNK
            pltpu.make_async_copy(m_hbm.at[pl.ds(base, CHUNK)], rm[b], sl[b]).wait()

        def sc_start(j, b):
            pltpu.async_copy(rm[b], acc_sh.at[slab.at[j, 0]], ss[b], add=True)

        def sc_wait(j, b):
            pltpu.make_async_copy(rm[b], acc_sh.at[slab.at[j, 0]], ss[b]).wait()

        l_start(0, 0)

        def body(j, carry):
            def step(b):
                l_wait(j, b)
                sc_start(j, b)
                nb = 1 - b

                @pl.when(j >= 1)
                def _():
                    sc_wait(j - 1, nb)

                @pl.when(j + 1 < WCHS)
                def _():
                    l_start(j + 1, nb)

            @pl.when(lax.rem(j, 2) == 0)
            def _():
                step(0)

            @pl.when(lax.rem(j, 2) == 1)
            def _():
                step(1)

            return carry

        lax.fori_loop(0, WCHS, body, 0)
        sc_wait(WCHS - 1, (WCHS - 1) % 2)

        @pl.when(wid_local < NREMS)
        def _():
            cr = WCHS * (NW // 2) + wid_local
            pltpu.sync_copy(m_hbm.at[pl.ds(cr * CHUNK, CHUNK)], rm0)
            pltpu.sync_copy(dst2_hbm.at[c_half + cr], slab.at[WCHS])
            pltpu.sync_copy(rm0, acc_sh.at[slab.at[WCHS, 0]], add=True)

    @pl.when(wid < NW // 2)
    def _():
        run_half(ma_hbm, 0, wid)

    @pl.when(wid >= NW // 2)
    def _():
        run_half(mb_hbm, NCH_H, wid - NW // 2)

    plsc.subcore_barrier()

    @pl.when(sid < NS - 1)
    def _():
        pltpu.sync_copy(acc_sh.at[pl.ds(r0, STRIPE)],
                        out_hbm.at[cid].at[pl.ds(r0, STRIPE)])

    @pl.when(sid == NS - 1)
    def _():
        pltpu.sync_copy(acc_sh.at[pl.ds(r0, STRIPE_LAST)],
                        out_hbm.at[cid].at[pl.ds(r0, STRIPE_LAST)])


def _ln(m, g, b):
    mu = jnp.mean(m, axis=-1, keepdims=True)
    var = jnp.mean((m - mu) ** 2, axis=-1, keepdims=True)
    return g * (m - mu) * lax.rsqrt(var + 1e-5) + b


def _enc_node_body(x_ref, we, be, ws0, wd0, oh, opq):
    h = jnp.dot(x_ref[...], we[...], preferred_element_type=jnp.float32) + be[...]
    oh[...] = h
    opq[0, :, :] = jnp.dot(h, ws0[...], preferred_element_type=jnp.float32)
    opq[1, :, :] = jnp.dot(h, wd0[...], preferred_element_type=jnp.float32)


def _enc_edge_body(a_ref, we, be, oe):
    oe[...] = jnp.dot(a_ref[...], we[...], preferred_element_type=jnp.float32) + be[...]


def _edge_body(e_ref, g1_ref, g2_ref, w1e, b1, w2, b2, lng, lnb, om):
    t = jnp.dot(e_ref[...], w1e[...], preferred_element_type=jnp.float32)
    t = t + g1_ref[...] + g2_ref[...] + b1[...]
    t = jnp.maximum(t, 0.0)
    m = jnp.dot(t, w2[...], preferred_element_type=jnp.float32) + b2[...]
    m = jnp.maximum(m, 0.0)
    om[...] = _ln(m, lng[...], lnb[...])


def _node_body(h_ref, p0_ref, p1_ref, u1h, u1p, bu1, u2, bu2, lng, lnb,
               wsn, wdn, oh, opq):
    pooled = p0_ref[...] + p1_ref[...]
    t = (jnp.dot(h_ref[...], u1h[...], preferred_element_type=jnp.float32)
         + jnp.dot(pooled, u1p[...], preferred_element_type=jnp.float32)
         + bu1[...])
    t = jnp.maximum(t, 0.0)
    u = jnp.dot(t, u2[...], preferred_element_type=jnp.float32) + bu2[...]
    u = jnp.maximum(u, 0.0)
    hn = h_ref[...] + _ln(u, lng[...], lnb[...])
    oh[...] = hn
    opq[0, :, :] = jnp.dot(hn, wsn[...], preferred_element_type=jnp.float32)
    opq[1, :, :] = jnp.dot(hn, wdn[...], preferred_element_type=jnp.float32)


def _node_final_body(h_ref, p0_ref, p1_ref, u1h, u1p, bu1, u2, bu2, lng, lnb,
                     wdec, bdec, oy):
    pooled = p0_ref[...] + p1_ref[...]
    t = (jnp.dot(h_ref[...], u1h[...], preferred_element_type=jnp.float32)
         + jnp.dot(pooled, u1p[...], preferred_element_type=jnp.float32)
         + bu1[...])
    t = jnp.maximum(t, 0.0)
    u = jnp.dot(t, u2[...], preferred_element_type=jnp.float32) + bu2[...]
    u = jnp.maximum(u, 0.0)
    hn = h_ref[...] + _ln(u, lng[...], lnb[...])
    oy[...] = jnp.dot(hn, wdec[...], preferred_element_type=jnp.float32) + bdec[...]


def _row_spec(bm, width):
    return pl.BlockSpec((bm, width), lambda i: (i, 0))


def _full_spec(shape):
    return pl.BlockSpec(shape, lambda i: tuple(0 for _ in shape))


BN = 2000
BE = 4000


_PQ_SPEC = pl.BlockSpec((2, BN, LATW), lambda i: (0, i, 0))


def _enc_node_call(x, we, be, ws0, wd0):
    return pl.pallas_call(
        _enc_node_body,
        grid=(N // BN,),
        in_specs=[_row_spec(BN, 256), _full_spec((256, LATENT)),
                  _full_spec((1, LATENT)), _full_spec((LATENT, LATENT)),
                  _full_spec((LATENT, LATENT))],
        out_specs=[_row_spec(BN, LATENT), _PQ_SPEC],
        out_shape=[jax.ShapeDtypeStruct((N, LATENT), jnp.float32),
                   jax.ShapeDtypeStruct((2, N, LATW), jnp.float32)],
    )(x, we, be, ws0, wd0)


def _enc_edge_call(attr, we, be):
    return pl.pallas_call(
        _enc_edge_body,
        grid=(E // BE,),
        in_specs=[_row_spec(BE, 16), _full_spec((16, LATENT)),
                  _full_spec((1, LATENT))],
        out_specs=_row_spec(BE, LATENT),
        out_shape=jax.ShapeDtypeStruct((E, LATENT), jnp.float32),
    )(attr, we, be)


def _edge_call(e, g1, g2, w1e, b1, w2, b2, lng, lnb, off_blocks):
    w = _full_spec((LATENT, LATENT))
    v = _full_spec((1, LATENT))
    e_spec = pl.BlockSpec((BE, LATENT), lambda i: (i + off_blocks, 0))
    return pl.pallas_call(
        _edge_body,
        grid=(EH // BE,),
        in_specs=[e_spec, _row_spec(BE, LATENT), _row_spec(BE, LATENT),
                  w, v, w, v, v, v],
        out_specs=_row_spec(BE, LATENT),
        out_shape=jax.ShapeDtypeStruct((EH, LATENT), jnp.float32),
    )(e, g1, g2, w1e, b1, w2, b2, lng, lnb)


def _node_call(h, p0, p1, u1h, u1p, bu1, u2, bu2, lng, lnb, wsn, wdn):
    w = _full_spec((LATENT, LATENT))
    v = _full_spec((1, LATENT))
    return pl.pallas_call(
        _node_body,
        grid=(N // BN,),
        in_specs=[_row_spec(BN, LATENT)] * 3 + [w, w, v, w, v, v, v, w, w],
        out_specs=[_row_spec(BN, LATENT), _PQ_SPEC],
        out_shape=[jax.ShapeDtypeStruct((N, LATENT), jnp.float32),
                   jax.ShapeDtypeStruct((2, N, LATW), jnp.float32)],
    )(h, p0, p1, u1h, u1p, bu1, u2, bu2, lng, lnb, wsn, wdn)


def _node_final_call(h, p0, p1, u1h, u1p, bu1, u2, bu2, lng, lnb, wdec, bdec):
    w = _full_spec((LATENT, LATENT))
    v = _full_spec((1, LATENT))
    return pl.pallas_call(
        _node_final_body,
        grid=(N // BN,),
        in_specs=[_row_spec(BN, LATENT)] * 3 + [w, w, v, w, v, v, v, w, v],
        out_specs=_row_spec(BN, LATENT),
        out_shape=jax.ShapeDtypeStruct((N, LATENT), jnp.float32),
    )(h, p0, p1, u1h, u1p, bu1, u2, bu2, lng, lnb, wdec, bdec)


def kernel(x, edge_index, edge_attr, params):
    src = edge_index[0]
    dst = edge_index[1]
    dst2 = dst.reshape(NCHUNKS, 1, CHUNK)
    sd = jnp.stack([src, dst + N], axis=0).reshape(2, NCHUNKS, CHUNK)
    sd = sd.transpose(1, 0, 2)
    layers = params["layers"]

    def row(v):
        return v.reshape(1, -1)

    w1 = [lp["msg1"]["w"] for lp in layers]
    w1e = [w[0:LATENT] for w in w1]
    w1s = [w[LATENT:2 * LATENT] for w in w1]
    w1d = [w[2 * LATENT:3 * LATENT] for w in w1]
    u1 = [lp["upd1"]["w"] for lp in layers]
    u1h = [w[0:LATENT] for w in u1]
    u1p = [w[LATENT:2 * LATENT] for w in u1]

    wdec = jnp.zeros((LATENT, LATENT), jnp.float32).at[:, :3].set(params["dec"]["w"])
    bdec = jnp.zeros((1, LATENT), jnp.float32).at[:, :3].set(params["dec"]["b"])
    zeros_n = jnp.zeros((N, LATENT), jnp.float32)

    h, pq = _enc_node_call(x, params["enc_node"]["w"], row(params["enc_node"]["b"]),
                           w1s[0], w1d[0])
    e = _enc_edge_call(edge_attr, params["enc_edge"]["w"], row(params["enc_edge"]["b"]))

    for l in range(NUM_MP):
        lp = layers[l]
        pqf = pq.reshape(2 * N, LATENT)
        g1a, g2a = _sc_gather_a(pqf, sd)
        g1b, g2b = _sc_gather_b(pqf, sd)
        ew = (w1e[l], row(lp["msg1"]["b"]), lp["msg2"]["w"], row(lp["msg2"]["b"]),
              row(lp["msg_ln_g"]), row(lp["msg_ln_b"]))
        ma = _edge_call(e, g1a, g2a, *ew, 0)
        mb = _edge_call(e, g1b, g2b, *ew, EH // BE)
        pooled2 = _sc_scatter(ma, mb, dst2, zeros_n)
        args = (h, pooled2[0], pooled2[1], u1h[l], u1p[l], row(lp["upd1"]["b"]),
                lp["upd2"]["w"], row(lp["upd2"]["b"]),
                row(lp["upd_ln_g"]), row(lp["upd_ln_b"]))
        if l < NUM_MP - 1:
            h, pq = _node_call(*args, w1s[l + 1], w1d[l + 1])
        else:
            y = _node_final_call(*args, wdec, bdec)

    return y[:, :3]


# speedup vs baseline: 1.0001x; 1.0001x over previous
import functools

import jax
import jax.numpy as jnp
from jax import lax
from jax.experimental import pallas as pl
from jax.experimental.pallas import tpu as pltpu
from jax.experimental.pallas import tpu_sc as plsc

N = 10000
E = 160000
LATENT = 128
NUM_MP = 10

CHUNK = 128
NCHUNKS = E // CHUNK
NC, NS = 2, 16
NW = NC * NS
STRIPE = 640
STRIPE_LAST = N - 15 * STRIPE

_mesh = plsc.VectorSubcoreMesh(core_axis_name="c", subcore_axis_name="s")


LATW = LATENT


CHG = 256
NCHG = E // CHG
WCHG = NCHG // (NW // 2)
NREMG = NCHG - WCHG * (NW // 2)


@functools.partial(
    pl.kernel,
    out_type=(jax.ShapeDtypeStruct((E, LATENT), jnp.float32),
              jax.ShapeDtypeStruct((E, LATENT), jnp.float32)),
    mesh=_mesh,
    scratch_types=[
        pltpu.VMEM((WCHG + 1, 1, CHG), jnp.int32),
        pltpu.VMEM((CHG, LATENT), jnp.float32),
        pltpu.VMEM((CHG, LATENT), jnp.float32),
        pltpu.VMEM((CHG, LATENT), jnp.float32),
        pltpu.SemaphoreType.DMA,
        pltpu.SemaphoreType.DMA,
        pltpu.SemaphoreType.DMA,
        pltpu.SemaphoreType.DMA,
        pltpu.SemaphoreType.DMA,
        pltpu.SemaphoreType.DMA,
    ],
)
def _sc_gather(pq_hbm, sidx_hbm, didx_hbm, g1_hbm, g2_hbm,
               slab, r0, r1, r2, sg0, sg1, sg2, so0, so1, so2):
    wid = lax.axis_index("s") * NC + lax.axis_index("c")
    rr = (r0, r1, r2)
    sg = (sg0, sg1, sg2)
    so = (so0, so1, so2)

    def run(idx_hbm, out_hbm, lid):
        c0 = lid * WCHG

        pltpu.sync_copy(idx_hbm.at[pl.ds(c0, WCHG)], slab.at[pl.ds(0, WCHG)])

        def g_start(j, b):
            pltpu.async_copy(pq_hbm.at[slab.at[j, 0]], rr[b], sg[b])

        def g_wait(j, b):
            pltpu.make_async_copy(pq_hbm.at[slab.at[j, 0]], rr[b], sg[b]).wait()

        def s_start(j, b):
            base = (c0 + j) * CHG
            pltpu.async_copy(rr[b], out_hbm.at[pl.ds(base, CHG)], so[b])

        def s_wait(j, b):
            base = (c0 + j) * CHG
            pltpu.make_async_copy(rr[b], out_hbm.at[pl.ds(base, CHG)], so[b]).wait()

        g_start(0, 0)
        g_start(1, 1)

        def body(j, carry):
            def step(b):
                g_wait(j, b)
                s_start(j, b)
                prev = (b + 2) % 3

                @pl.when(j >= 1)
                def _():
                    s_wait(j - 1, prev)

                @pl.when(j + 2 < WCHG)
                def _():
                    g_start(j + 2, prev)

            for k in range(3):
                @pl.when(lax.rem(j, 3) == k)
                def _(k=k):
                    step(k)

            return carry

        lax.fori_loop(0, WCHG, body, 0)
        s_wait(WCHG - 1, (WCHG - 1) % 3)

        @pl.when(lid < NREMG)
        def _():
            cr = WCHG * (NW // 2) + lid
            pltpu.sync_copy(idx_hbm.at[cr], slab.at[WCHG])
            pltpu.async_copy(pq_hbm.at[slab.at[WCHG, 0]], rr[0], sg[0])
            pltpu.make_async_copy(pq_hbm.at[slab.at[WCHG, 0]], rr[0], sg[0]).wait()
            pltpu.sync_copy(rr[0], out_hbm.at[pl.ds(cr * CHG, CHG)])

    @pl.when(wid < NW // 2)
    def _():
        run(sidx_hbm, g1_hbm, wid)

    @pl.when(wid >= NW // 2)
    def _():
        run(didx_hbm, g2_hbm, wid - NW // 2)


NCH_H = NCHUNKS // 2
EH = NCH_H * CHUNK
WCHS = NCH_H // (NW // 2)
NREMS = NCH_H - WCHS * (NW // 2)


@functools.partial(
    pl.kernel,
    out_type=jax.ShapeDtypeStruct((NC, N, LATENT), jnp.float32),
    mesh=_mesh,
    scratch_types=[
        pltpu.VMEM((WCHS + 1, 1, CHUNK), jnp.int32),
        pltpu.VMEM((CHUNK, LATENT), jnp.float32),
        pltpu.VMEM((CHUNK, LATENT), jnp.float32),
        pltpu.VMEM_SHARED((N, LATENT), jnp.float32),
        pltpu.SemaphoreType.DMA,
        pltpu.SemaphoreType.DMA,
        pltpu.SemaphoreType.DMA,
        pltpu.SemaphoreType.DMA,
    ],
)
def _sc_scatter(ma_hbm, mb_hbm, dst2_hbm, zeros_hbm, out_hbm,
                slab, rm0, rm1, acc_sh, sl0, sl1, ss0, ss1):
    cid = lax.axis_index("c")
    sid = lax.axis_index("s")
    wid = sid * NC + cid
    r0 = sid * STRIPE
    rm = (rm0, rm1)
    sl = (sl0, sl1)
    ss = (ss0, ss1)

    @pl.when(sid < NS - 1)
    def _():
        pltpu.sync_copy(zeros_hbm.at[pl.ds(r0, STRIPE)],
                        acc_sh.at[pl.ds(r0, STRIPE)])

    @pl.when(sid == NS - 1)
    def _():
        pltpu.sync_copy(zeros_hbm.at[pl.ds(r0, STRIPE_LAST)],
                        acc_sh.at[pl.ds(r0, STRIPE_LAST)])

    plsc.subcore_barrier()

    def run_half(m_hbm, c_half, wid_local):
        c0 = wid_local * WCHS

        pltpu.sync_copy(dst2_hbm.at[pl.ds(c_half + c0, WCHS)],
                        slab.at[pl.ds(0, WCHS)])

        def l_start(j, b):
            base = (c0 + j) * CHUNK
            pltpu.async_copy(m_hbm.at[pl.ds(base, CHUNK)], rm[b], sl[b])

        def l_wait(j, b):
            base = (c0 + j) * CHUNK
            pltpu.make_async_copy(m_hbm.at[pl.ds(base, CHUNK)], rm[b], sl[b]).wait()

        def sc_start(j, b):
            pltpu.async_copy(rm[b], acc_sh.at[slab.at[j, 0]], ss[b], add=True)

        def sc_wait(j, b):
            pltpu.make_async_copy(rm[b], acc_sh.at[slab.at[j, 0]], ss[b]).wait()

        l_start(0, 0)

        def body(j, carry):
            def step(b):
                l_wait(j, b)
                sc_start(j, b)
                nb = 1 - b

                @pl.when(j >= 1)
                def _():
                    sc_wait(j - 1, nb)

                @pl.when(j + 1 < WCHS)
                def _():
                    l_start(j + 1, nb)

            @pl.when(lax.rem(j, 2) == 0)
            def _():
                step(0)

            @pl.when(lax.rem(j, 2) == 1)
            def _():
                step(1)

            return carry

        lax.fori_loop(0, WCHS, body, 0)
        sc_wait(WCHS - 1, (WCHS - 1) % 2)

        @pl.when(wid_local < NREMS)
        def _():
            cr = WCHS * (NW // 2) + wid_local
            pltpu.sync_copy(m_hbm.at[pl.ds(cr * CHUNK, CHUNK)], rm0)
            pltpu.sync_copy(dst2_hbm.at[c_half + cr], slab.at[WCHS])
            pltpu.sync_copy(rm0, acc_sh.at[slab.at[WCHS, 0]], add=True)

    @pl.when(wid < NW // 2)
    def _():
        run_half(ma_hbm, 0, wid)

    @pl.when(wid >= NW // 2)
    def _():
        run_half(mb_hbm, NCH_H, wid - NW // 2)

    plsc.subcore_barrier()

    @pl.when(sid < NS - 1)
    def _():
        pltpu.sync_copy(acc_sh.at[pl.ds(r0, STRIPE)],
                        out_hbm.at[cid].at[pl.ds(r0, STRIPE)])

    @pl.when(sid == NS - 1)
    def _():
        pltpu.sync_copy(acc_sh.at[pl.ds(r0, STRIPE_LAST)],
                        out_hbm.at[cid].at[pl.ds(r0, STRIPE_LAST)])


def _ln(m, g, b):
    mu = jnp.mean(m, axis=-1, keepdims=True)
    var = jnp.mean((m - mu) ** 2, axis=-1, keepdims=True)
    return g * (m - mu) * lax.rsqrt(var + 1e-5) + b


def _enc_node_body(x_ref, we, be, ws0, wd0, oh, opq):
    h = jnp.dot(x_ref[...], we[...], preferred_element_type=jnp.float32) + be[...]
    oh[...] = h
    opq[0, :, :] = jnp.dot(h, ws0[...], preferred_element_type=jnp.float32)
    opq[1, :, :] = jnp.dot(h, wd0[...], preferred_element_type=jnp.float32)


def _enc_edge_body(a_ref, we, be, oe):
    oe[...] = jnp.dot(a_ref[...], we[...], preferred_element_type=jnp.float32) + be[...]


def _edge_body(e_ref, g1_ref, g2_ref, w1e, b1, w2, b2, lng, lnb, om):
    t = jnp.dot(e_ref[...], w1e[...], preferred_element_type=jnp.float32)
    t = t + g1_ref[...] + g2_ref[...] + b1[...]
    t = jnp.maximum(t, 0.0)
    m = jnp.dot(t, w2[...], preferred_element_type=jnp.float32) + b2[...]
    m = jnp.maximum(m, 0.0)
    om[...] = _ln(m, lng[...], lnb[...])


def _node_body(h_ref, p0_ref, p1_ref, u1h, u1p, bu1, u2, bu2, lng, lnb,
               wsn, wdn, oh, opq):
    pooled = p0_ref[...] + p1_ref[...]
    t = (jnp.dot(h_ref[...], u1h[...], preferred_element_type=jnp.float32)
         + jnp.dot(pooled, u1p[...], preferred_element_type=jnp.float32)
         + bu1[...])
    t = jnp.maximum(t, 0.0)
    u = jnp.dot(t, u2[...], preferred_element_type=jnp.float32) + bu2[...]
    u = jnp.maximum(u, 0.0)
    hn = h_ref[...] + _ln(u, lng[...], lnb[...])
    oh[...] = hn
    opq[0, :, :] = jnp.dot(hn, wsn[...], preferred_element_type=jnp.float32)
    opq[1, :, :] = jnp.dot(hn, wdn[...], preferred_element_type=jnp.float32)


def _node_final_body(h_ref, p0_ref, p1_ref, u1h, u1p, bu1, u2, bu2, lng, lnb,
                     wdec, bdec, oy):
    pooled = p0_ref[...] + p1_ref[...]
    t = (jnp.dot(h_ref[...], u1h[...], preferred_element_type=jnp.float32)
         + jnp.dot(pooled, u1p[...], preferred_element_type=jnp.float32)
         + bu1[...])
    t = jnp.maximum(t, 0.0)
    u = jnp.dot(t, u2[...], preferred_element_type=jnp.float32) + bu2[...]
    u = jnp.maximum(u, 0.0)
    hn = h_ref[...] + _ln(u, lng[...], lnb[...])
    oy[...] = jnp.dot(hn, wdec[...], preferred_element_type=jnp.float32) + bdec[...]


def _row_spec(bm, width):
    return pl.BlockSpec((bm, width), lambda i: (i, 0))


def _full_spec(shape):
    return pl.BlockSpec(shape, lambda i: tuple(0 for _ in shape))


BN = 2000
BE = 4000


_PQ_SPEC = pl.BlockSpec((2, BN, LATW), lambda i: (0, i, 0))


def _enc_node_call(x, we, be, ws0, wd0):
    return pl.pallas_call(
        _enc_node_body,
        grid=(N // BN,),
        in_specs=[_row_spec(BN, 256), _full_spec((256, LATENT)),
                  _full_spec((1, LATENT)), _full_spec((LATENT, LATENT)),
                  _full_spec((LATENT, LATENT))],
        out_specs=[_row_spec(BN, LATENT), _PQ_SPEC],
        out_shape=[jax.ShapeDtypeStruct((N, LATENT), jnp.float32),
                   jax.ShapeDtypeStruct((2, N, LATW), jnp.float32)],
    )(x, we, be, ws0, wd0)


def _enc_edge_call(attr, we, be):
    return pl.pallas_call(
        _enc_edge_body,
        grid=(E // BE,),
        in_specs=[_row_spec(BE, 16), _full_spec((16, LATENT)),
                  _full_spec((1, LATENT))],
        out_specs=_row_spec(BE, LATENT),
        out_shape=jax.ShapeDtypeStruct((E, LATENT), jnp.float32),
    )(attr, we, be)


def _edge_call(e, g1, g2, w1e, b1, w2, b2, lng, lnb, off_blocks):
    w = _full_spec((LATENT, LATENT))
    v = _full_spec((1, LATENT))
    e_spec = pl.BlockSpec((BE, LATENT), lambda i: (i + off_blocks, 0))
    return pl.pallas_call(
        _edge_body,
        grid=(EH // BE,),
        in_specs=[e_spec, e_spec, e_spec, w, v, w, v, v, v],
        out_specs=_row_spec(BE, LATENT),
        out_shape=jax.ShapeDtypeStruct((EH, LATENT), jnp.float32),
    )(e, g1, g2, w1e, b1, w2, b2, lng, lnb)


def _node_call(h, p0, p1, u1h, u1p, bu1, u2, bu2, lng, lnb, wsn, wdn):
    w = _full_spec((LATENT, LATENT))
    v = _full_spec((1, LATENT))
    return pl.pallas_call(
        _node_body,
        grid=(N // BN,),
        in_specs=[_row_spec(BN, LATENT)] * 3 + [w, w, v, w, v, v, v, w, w],
        out_specs=[_row_spec(BN, LATENT), _PQ_SPEC],
        out_shape=[jax.ShapeDtypeStruct((N, LATENT), jnp.float32),
                   jax.ShapeDtypeStruct((2, N, LATW), jnp.float32)],
    )(h, p0, p1, u1h, u1p, bu1, u2, bu2, lng, lnb, wsn, wdn)


def _node_final_call(h, p0, p1, u1h, u1p, bu1, u2, bu2, lng, lnb, wdec, bdec):
    w = _full_spec((LATENT, LATENT))
    v = _full_spec((1, LATENT))
    return pl.pallas_call(
        _node_final_body,
        grid=(N // BN,),
        in_specs=[_row_spec(BN, LATENT)] * 3 + [w, w, v, w, v, v, v, w, v],
        out_specs=_row_spec(BN, LATENT),
        out_shape=jax.ShapeDtypeStruct((N, LATENT), jnp.float32),
    )(h, p0, p1, u1h, u1p, bu1, u2, bu2, lng, lnb, wdec, bdec)


def kernel(x, edge_index, edge_attr, params):
    src = edge_index[0]
    dst = edge_index[1]
    dst2 = dst.reshape(NCHUNKS, 1, CHUNK)
    sidx = src.reshape(NCHG, 1, CHG)
    didx = (dst + N).reshape(NCHG, 1, CHG)
    layers = params["layers"]

    def row(v):
        return v.reshape(1, -1)

    w1 = [lp["msg1"]["w"] for lp in layers]
    w1e = [w[0:LATENT] for w in w1]
    w1s = [w[LATENT:2 * LATENT] for w in w1]
    w1d = [w[2 * LATENT:3 * LATENT] for w in w1]
    u1 = [lp["upd1"]["w"] for lp in layers]
    u1h = [w[0:LATENT] for w in u1]
    u1p = [w[LATENT:2 * LATENT] for w in u1]

    wdec = jnp.zeros((LATENT, LATENT), jnp.float32).at[:, :3].set(params["dec"]["w"])
    bdec = jnp.zeros((1, LATENT), jnp.float32).at[:, :3].set(params["dec"]["b"])
    zeros_n = jnp.zeros((N, LATENT), jnp.float32)

    h, pq = _enc_node_call(x, params["enc_node"]["w"], row(params["enc_node"]["b"]),
                           w1s[0], w1d[0])
    e = _enc_edge_call(edge_attr, params["enc_edge"]["w"], row(params["enc_edge"]["b"]))

    for l in range(NUM_MP):
        lp = layers[l]
        pqf = pq.reshape(2 * N, LATENT)
        g1, g2 = _sc_gather(pqf, sidx, didx)
        ew = (w1e[l], row(lp["msg1"]["b"]), lp["msg2"]["w"], row(lp["msg2"]["b"]),
              row(lp["msg_ln_g"]), row(lp["msg_ln_b"]))
        ma = _edge_call(e, g1, g2, *ew, 0)
        mb = _edge_call(e, g1, g2, *ew, EH // BE)
        pooled2 = _sc_scatter(ma, mb, dst2, zeros_n)
        args = (h, pooled2[0], pooled2[1], u1h[l], u1p[l], row(lp["upd1"]["b"]),
                lp["upd2"]["w"], row(lp["upd2"]["b"]),
                row(lp["upd_ln_g"]), row(lp["upd_ln_b"]))
        if l < NUM_MP - 1:
            h, pq = _node_call(*args, w1s[l + 1], w1d[l + 1])
        else:
            y = _node_final_call(*args, wdec, bdec)

    return y[:, :3]


# speedup vs baseline: 1.1458x; 1.1456x over previous
import functools

import jax
import jax.numpy as jnp
from jax import lax
from jax.experimental import pallas as pl
from jax.experimental.pallas import tpu as pltpu
from jax.experimental.pallas import tpu_sc as plsc

N = 10000
E = 160000
LATENT = 128
NUM_MP = 10

CHUNK = 128
NCHUNKS = E // CHUNK
NC, NS = 2, 16
NW = NC * NS
STRIPE = 640
STRIPE_LAST = N - 15 * STRIPE

_mesh = plsc.VectorSubcoreMesh(core_axis_name="c", subcore_axis_name="s")


LATW = LATENT


def _make_gather(c_lo, nch):
    eh = nch * CHUNK
    wch = nch // NW
    nrem = nch - wch * NW

    @functools.partial(
        pl.kernel,
        out_type=jax.ShapeDtypeStruct((eh, LATENT), jnp.float32),
        mesh=_mesh,
        scratch_types=[
            pltpu.VMEM((wch + 1, 2, CHUNK), jnp.int32),
            pltpu.VMEM((CHUNK, LATENT), jnp.float32),
            pltpu.VMEM((CHUNK, LATENT), jnp.float32),
            pltpu.VMEM((CHUNK, LATENT), jnp.float32),
            pltpu.VMEM((CHUNK, LATENT), jnp.float32),
            pltpu.VMEM((CHUNK, LATENT), jnp.float32),
            pltpu.VMEM((CHUNK, LATENT), jnp.float32),
            pltpu.SemaphoreType.DMA,
            pltpu.SemaphoreType.DMA,
            pltpu.SemaphoreType.DMA,
            pltpu.SemaphoreType.DMA,
            pltpu.SemaphoreType.DMA,
            pltpu.SemaphoreType.DMA,
        ],
    )
    def gather(pq_hbm, sd_hbm, g_hbm, slab,
               rp0, rp1, rp2, rq0, rq1, rq2, sg0, sg1, sg2, so0, so1, so2):
        wid = lax.axis_index("s") * NC + lax.axis_index("c")
        c0 = wid * wch
        rp = (rp0, rp1, rp2)
        rq = (rq0, rq1, rq2)
        sg = (sg0, sg1, sg2)
        so = (so0, so1, so2)

        pltpu.sync_copy(sd_hbm.at[pl.ds(c_lo + c0, wch)],
                        slab.at[pl.ds(0, wch)])

        def g_start(j, b):
            pltpu.async_copy(pq_hbm.at[slab.at[j, 0]], rp[b], sg[b])
            pltpu.async_copy(pq_hbm.at[slab.at[j, 1]], rq[b], sg[b])

        def g_wait(j, b):
            pltpu.make_async_copy(pq_hbm.at[slab.at[j, 0]], rp[b], sg[b]).wait()
            pltpu.make_async_copy(pq_hbm.at[slab.at[j, 1]], rq[b], sg[b]).wait()

        def s_start(j, b):
            base = (c0 + j) * CHUNK
            pltpu.async_copy(rp[b], g_hbm.at[pl.ds(base, CHUNK)], so[b])

        def s_wait(j, b):
            base = (c0 + j) * CHUNK
            pltpu.make_async_copy(rp[b], g_hbm.at[pl.ds(base, CHUNK)], so[b]).wait()

        def vadd(b):
            def rowbody(r, carry):
                for k in range(LATENT // 16):
                    cs = pl.ds(k * 16, 16)
                    rp[b][r, cs] = rp[b][r, cs] + rq[b][r, cs]
                return carry

            lax.fori_loop(0, CHUNK, rowbody, 0)

        g_start(0, 0)
        g_start(1, 1)

        def body(j, carry):
            def step(b):
                g_wait(j, b)
                vadd(b)
                s_start(j, b)
                prev = (b + 2) % 3

                @pl.when(j >= 1)
                def _():
                    s_wait(j - 1, prev)

                @pl.when(j + 2 < wch)
                def _():
                    g_start(j + 2, prev)

            for k in range(3):
                @pl.when(lax.rem(j, 3) == k)
                def _(k=k):
                    step(k)

            return carry

        lax.fori_loop(0, wch, body, 0)
        s_wait(wch - 1, (wch - 1) % 3)

        @pl.when(wid < nrem)
        def _():
            cr = wch * NW + wid
            pltpu.sync_copy(sd_hbm.at[c_lo + cr], slab.at[wch])
            pltpu.async_copy(pq_hbm.at[slab.at[wch, 0]], rp0, sg0)
            pltpu.async_copy(pq_hbm.at[slab.at[wch, 1]], rq0, sg0)
            pltpu.make_async_copy(pq_hbm.at[slab.at[wch, 0]], rp0, sg0).wait()
            pltpu.make_async_copy(pq_hbm.at[slab.at[wch, 1]], rq0, sg0).wait()

            def rowbody(r, carry):
                for k in range(LATENT // 16):
                    cs = pl.ds(k * 16, 16)
                    rp0[r, cs] = rp0[r, cs] + rq0[r, cs]
                return carry

            lax.fori_loop(0, CHUNK, rowbody, 0)
            pltpu.sync_copy(rp0, g_hbm.at[pl.ds(cr * CHUNK, CHUNK)])

    return gather


NCH_H = NCHUNKS // 2
EH = NCH_H * CHUNK
_sc_gather_a = _make_gather(0, NCH_H)
_sc_gather_b = _make_gather(NCH_H, NCH_H)


WCHS = NCH_H // (NW // 2)
NREMS = NCH_H - WCHS * (NW // 2)


@functools.partial(
    pl.kernel,
    out_type=jax.ShapeDtypeStruct((NC, N, LATENT), jnp.float32),
    mesh=_mesh,
    scratch_types=[
        pltpu.VMEM((WCHS + 1, 1, CHUNK), jnp.int32),
        pltpu.VMEM((CHUNK, LATENT), jnp.float32),
        pltpu.VMEM((CHUNK, LATENT), jnp.float32),
        pltpu.VMEM_SHARED((N, LATENT), jnp.float32),
        pltpu.SemaphoreType.DMA,
        pltpu.SemaphoreType.DMA,
        pltpu.SemaphoreType.DMA,
        pltpu.SemaphoreType.DMA,
    ],
)
def _sc_scatter(ma_hbm, mb_hbm, dst2_hbm, zeros_hbm, out_hbm,
                slab, rm0, rm1, acc_sh, sl0, sl1, ss0, ss1):
    cid = lax.axis_index("c")
    sid = lax.axis_index("s")
    wid = sid * NC + cid
    r0 = sid * STRIPE
    rm = (rm0, rm1)
    sl = (sl0, sl1)
    ss = (ss0, ss1)

    @pl.when(sid < NS - 1)
    def _():
        pltpu.sync_copy(zeros_hbm.at[pl.ds(r0, STRIPE)],
                        acc_sh.at[pl.ds(r0, STRIPE)])

    @pl.when(sid == NS - 1)
    def _():
        pltpu.sync_copy(zeros_hbm.at[pl.ds(r0, STRIPE_LAST)],
                        acc_sh.at[pl.ds(r0, STRIPE_LAST)])

    plsc.subcore_barrier()

    def run_half(m_hbm, c_half, wid_local):
        c0 = wid_local * WCHS

        pltpu.sync_copy(dst2_hbm.at[pl.ds(c_half + c0, WCHS)],
                        slab.at[pl.ds(0, WCHS)])

        def l_start(j, b):
            base = (c0 + j) * CHUNK
            pltpu.async_copy(m_hbm.at[pl.ds(base, CHUNK)], rm[b], sl[b])

        def l_wait(j, b):
            base = (c0 + j) * CHUNK
            pltpu.make_async_copy(m_hbm.at[pl.ds(base, CHUNK)], rm[b], sl[b]).wait()

        def sc_start(j, b):
            pltpu.async_copy(rm[b], acc_sh.at[slab.at[j, 0]], ss[b], add=True)

        def sc_wait(j, b):
            pltpu.make_async_copy(rm[b], acc_sh.at[slab.at[j, 0]], ss[b]).wait()

        l_start(0, 0)

        def body(j, carry):
            def step(b):
                l_wait(j, b)
                sc_start(j, b)
                nb = 1 - b

                @pl.when(j >= 1)
                def _():
                    sc_wait(j - 1, nb)

                @pl.when(j + 1 < WCHS)
                def _():
                    l_start(j + 1, nb)

            @pl.when(lax.rem(j, 2) == 0)
            def _():
                step(0)

            @pl.when(lax.rem(j, 2) == 1)
            def _():
                step(1)

            return carry

        lax.fori_loop(0, WCHS, body, 0)
        sc_wait(WCHS - 1, (WCHS - 1) % 2)

        @pl.when(wid_local < NREMS)
        def _():
            cr = WCHS * (NW // 2) + wid_local
            pltpu.sync_copy(m_hbm.at[pl.ds(cr * CHUNK, CHUNK)], rm0)
            pltpu.sync_copy(dst2_hbm.at[c_half + cr], slab.at[WCHS])
            pltpu.sync_copy(rm0, acc_sh.at[slab.at[WCHS, 0]], add=True)

    @pl.when(wid < NW // 2)
    def _():
        run_half(ma_hbm, 0, wid)

    @pl.when(wid >= NW // 2)
    def _():
        run_half(mb_hbm, NCH_H, wid - NW // 2)

    plsc.subcore_barrier()

    @pl.when(sid < NS - 1)
    def _():
        pltpu.sync_copy(acc_sh.at[pl.ds(r0, STRIPE)],
                        out_hbm.at[cid].at[pl.ds(r0, STRIPE)])

    @pl.when(sid == NS - 1)
    def _():
        pltpu.sync_copy(acc_sh.at[pl.ds(r0, STRIPE_LAST)],
                        out_hbm.at[cid].at[pl.ds(r0, STRIPE_LAST)])


def _ln(m, g, b):
    mu = jnp.mean(m, axis=-1, keepdims=True)
    var = jnp.mean((m - mu) ** 2, axis=-1, keepdims=True)
    return g * (m - mu) * lax.rsqrt(var + 1e-5) + b


def _enc_node_body(x_ref, we, be, ws0, wd0, oh, opq):
    h = jnp.dot(x_ref[...], we[...], preferred_element_type=jnp.float32) + be[...]
    oh[...] = h
    opq[0, :, :] = jnp.dot(h, ws0[...], preferred_element_type=jnp.float32)
    opq[1, :, :] = jnp.dot(h, wd0[...], preferred_element_type=jnp.float32)


def _enc_edge_body(a_ref, we, be, oe):
    oe[...] = jnp.dot(a_ref[...], we[...], preferred_element_type=jnp.float32) + be[...]


def _edge_body(e_ref, g_ref, w1e, b1, w2, b2, lng, lnb, om):
    t = jnp.dot(e_ref[...], w1e[...], preferred_element_type=jnp.float32)
    t = t + g_ref[...] + b1[...]
    t = jnp.maximum(t, 0.0)
    m = jnp.dot(t, w2[...], preferred_element_type=jnp.float32) + b2[...]
    m = jnp.maximum(m, 0.0)
    om[...] = _ln(m, lng[...], lnb[...])


def _node_body(h_ref, p0_ref, p1_ref, u1h, u1p, bu1, u2, bu2, lng, lnb,
               wsn, wdn, oh, opq):
    pooled = p0_ref[...] + p1_ref[...]
    t = (jnp.dot(h_ref[...], u1h[...], preferred_element_type=jnp.float32)
         + jnp.dot(pooled, u1p[...], preferred_element_type=jnp.float32)
         + bu1[...])
    t = jnp.maximum(t, 0.0)
    u = jnp.dot(t, u2[...], preferred_element_type=jnp.float32) + bu2[...]
    u = jnp.maximum(u, 0.0)
    hn = h_ref[...] + _ln(u, lng[...], lnb[...])
    oh[...] = hn
    opq[0, :, :] = jnp.dot(hn, wsn[...], preferred_element_type=jnp.float32)
    opq[1, :, :] = jnp.dot(hn, wdn[...], preferred_element_type=jnp.float32)


def _node_final_body(h_ref, p0_ref, p1_ref, u1h, u1p, bu1, u2, bu2, lng, lnb,
                     wdec, bdec, oy):
    pooled = p0_ref[...] + p1_ref[...]
    t = (jnp.dot(h_ref[...], u1h[...], preferred_element_type=jnp.float32)
         + jnp.dot(pooled, u1p[...], preferred_element_type=jnp.float32)
         + bu1[...])
    t = jnp.maximum(t, 0.0)
    u = jnp.dot(t, u2[...], preferred_element_type=jnp.float32) + bu2[...]
    u = jnp.maximum(u, 0.0)
    hn = h_ref[...] + _ln(u, lng[...], lnb[...])
    oy[...] = jnp.dot(hn, wdec[...], preferred_element_type=jnp.float32) + bdec[...]


def _row_spec(bm, width):
    return pl.BlockSpec((bm, width), lambda i: (i, 0))


def _full_spec(shape):
    return pl.BlockSpec(shape, lambda i: tuple(0 for _ in shape))


BN = 2000
BE = 4000


_PQ_SPEC = pl.BlockSpec((2, BN, LATW), lambda i: (0, i, 0))


def _enc_node_call(x, we, be, ws0, wd0):
    return pl.pallas_call(
        _enc_node_body,
        grid=(N // BN,),
        in_specs=[_row_spec(BN, 256), _full_spec((256, LATENT)),
                  _full_spec((1, LATENT)), _full_spec((LATENT, LATENT)),
                  _full_spec((LATENT, LATENT))],
        out_specs=[_row_spec(BN, LATENT), _PQ_SPEC],
        out_shape=[jax.ShapeDtypeStruct((N, LATENT), jnp.float32),
                   jax.ShapeDtypeStruct((2, N, LATW), jnp.float32)],
    )(x, we, be, ws0, wd0)


def _enc_edge_call(attr, we, be):
    return pl.pallas_call(
        _enc_edge_body,
        grid=(E // BE,),
        in_specs=[_row_spec(BE, 16), _full_spec((16, LATENT)),
                  _full_spec((1, LATENT))],
        out_specs=_row_spec(BE, LATENT),
        out_shape=jax.ShapeDtypeStruct((E, LATENT), jnp.float32),
    )(attr, we, be)


def _edge_call(e, g, w1e, b1, w2, b2, lng, lnb, off_blocks):
    w = _full_spec((LATENT, LATENT))
    v = _full_spec((1, LATENT))
    e_spec = pl.BlockSpec((BE, LATENT), lambda i: (i + off_blocks, 0))
    return pl.pallas_call(
        _edge_body,
        grid=(EH // BE,),
        in_specs=[e_spec, _row_spec(BE, LATENT), w, v, w, v, v, v],
        out_specs=_row_spec(BE, LATENT),
        out_shape=jax.ShapeDtypeStruct((EH, LATENT), jnp.float32),
    )(e, g, w1e, b1, w2, b2, lng, lnb)


def _node_call(h, p0, p1, u1h, u1p, bu1, u2, bu2, lng, lnb, wsn, wdn):
    w = _full_spec((LATENT, LATENT))
    v = _full_spec((1, LATENT))
    return pl.pallas_call(
        _node_body,
        grid=(N // BN,),
        in_specs=[_row_spec(BN, LATENT)] * 3 + [w, w, v, w, v, v, v, w, w],
        out_specs=[_row_spec(BN, LATENT), _PQ_SPEC],
        out_shape=[jax.ShapeDtypeStruct((N, LATENT), jnp.float32),
                   jax.ShapeDtypeStruct((2, N, LATW), jnp.float32)],
    )(h, p0, p1, u1h, u1p, bu1, u2, bu2, lng, lnb, wsn, wdn)


def _node_final_call(h, p0, p1, u1h, u1p, bu1, u2, bu2, lng, lnb, wdec, bdec):
    w = _full_spec((LATENT, LATENT))
    v = _full_spec((1, LATENT))
    return pl.pallas_call(
        _node_final_body,
        grid=(N // BN,),
        in_specs=[_row_spec(BN, LATENT)] * 3 + [w, w, v, w, v, v, v, w, v],
        out_specs=_row_spec(BN, LATENT),
        out_shape=jax.ShapeDtypeStruct((N, LATENT), jnp.float32),
    )(h, p0, p1, u1h, u1p, bu1, u2, bu2, lng, lnb, wdec, bdec)


def kernel(x, edge_index, edge_attr, params):
    src = edge_index[0]
    dst = edge_index[1]
    dst2 = dst.reshape(NCHUNKS, 1, CHUNK)
    sd = jnp.stack([src, dst + N], axis=0).reshape(2, NCHUNKS, CHUNK)
    sd = sd.transpose(1, 0, 2)
    layers = params["layers"]

    def row(v):
        return v.reshape(1, -1)

    w1 = [lp["msg1"]["w"] for lp in layers]
    w1e = [w[0:LATENT] for w in w1]
    w1s = [w[LATENT:2 * LATENT] for w in w1]
    w1d = [w[2 * LATENT:3 * LATENT] for w in w1]
    u1 = [lp["upd1"]["w"] for lp in layers]
    u1h = [w[0:LATENT] for w in u1]
    u1p = [w[LATENT:2 * LATENT] for w in u1]

    wdec = jnp.zeros((LATENT, LATENT), jnp.float32).at[:, :3].set(params["dec"]["w"])
    bdec = jnp.zeros((1, LATENT), jnp.float32).at[:, :3].set(params["dec"]["b"])
    zeros_n = jnp.zeros((N, LATENT), jnp.float32)

    h, pq = _enc_node_call(x, params["enc_node"]["w"], row(params["enc_node"]["b"]),
                           w1s[0], w1d[0])
    e = _enc_edge_call(edge_attr, params["enc_edge"]["w"], row(params["enc_edge"]["b"]))

    for l in range(NUM_MP):
        lp = layers[l]
        pqf = pq.reshape(2 * N, LATENT)
        ga = _sc_gather_a(pqf, sd)
        gb = _sc_gather_b(pqf, sd)
        ew = (w1e[l], row(lp["msg1"]["b"]), lp["msg2"]["w"], row(lp["msg2"]["b"]),
              row(lp["msg_ln_g"]), row(lp["msg_ln_b"]))
        ma = _edge_call(e, ga, *ew, 0)
        mb = _edge_call(e, gb, *ew, EH // BE)
        pooled2 = _sc_scatter(ma, mb, dst2, zeros_n)
        args = (h, pooled2[0], pooled2[1], u1h[l], u1p[l], row(lp["upd1"]["b"]),
                lp["upd2"]["w"], row(lp["upd2"]["b"]),
                row(lp["upd_ln_g"]), row(lp["upd_ln_b"]))
        if l < NUM_MP - 1:
            h, pq = _node_call(*args, w1s[l + 1], w1d[l + 1])
        else:
            y = _node_final_call(*args, wdec, bdec)

    return y[:, :3]


# speedup vs baseline: 1.1867x; 1.0356x over previous
import functools

import jax
import jax.numpy as jnp
from jax import lax
from jax.experimental import pallas as pl
from jax.experimental.pallas import tpu as pltpu
from jax.experimental.pallas import tpu_sc as plsc

N = 10000
E = 160000
LATENT = 128
NUM_MP = 10

CHUNK = 128
NCHUNKS = E // CHUNK
NC, NS = 2, 16
NW = NC * NS
STRIPE = 640
STRIPE_LAST = N - 15 * STRIPE

_mesh = plsc.VectorSubcoreMesh(core_axis_name="c", subcore_axis_name="s")


LATW = LATENT


def _make_gather(c_lo, nch):
    eh = nch * CHUNK
    wch = nch // NW
    nrem = nch - wch * NW

    @functools.partial(
        pl.kernel,
        out_type=jax.ShapeDtypeStruct((eh, LATENT), jnp.float32),
        mesh=_mesh,
        scratch_types=[
            pltpu.VMEM((wch + 1, 2, CHUNK), jnp.int32),
            pltpu.VMEM((CHUNK, LATENT), jnp.float32),
            pltpu.VMEM((CHUNK, LATENT), jnp.float32),
            pltpu.VMEM((CHUNK, LATENT), jnp.float32),
            pltpu.VMEM((CHUNK, LATENT), jnp.float32),
            pltpu.VMEM((CHUNK, LATENT), jnp.float32),
            pltpu.VMEM((CHUNK, LATENT), jnp.float32),
            pltpu.SemaphoreType.DMA,
            pltpu.SemaphoreType.DMA,
            pltpu.SemaphoreType.DMA,
            pltpu.SemaphoreType.DMA,
            pltpu.SemaphoreType.DMA,
            pltpu.SemaphoreType.DMA,
        ],
    )
    def gather(pq_hbm, sd_hbm, g_hbm, slab,
               rp0, rp1, rp2, rq0, rq1, rq2, sg0, sg1, sg2, so0, so1, so2):
        wid = lax.axis_index("s") * NC + lax.axis_index("c")
        c0 = wid * wch
        rp = (rp0, rp1, rp2)
        rq = (rq0, rq1, rq2)
        sg = (sg0, sg1, sg2)
        so = (so0, so1, so2)

        pltpu.sync_copy(sd_hbm.at[pl.ds(c_lo + c0, wch)],
                        slab.at[pl.ds(0, wch)])

        def g_start(j, b):
            pltpu.async_copy(pq_hbm.at[slab.at[j, 0]], rp[b], sg[b])
            pltpu.async_copy(pq_hbm.at[slab.at[j, 1]], rq[b], sg[b])

        def g_wait(j, b):
            pltpu.make_async_copy(pq_hbm.at[slab.at[j, 0]], rp[b], sg[b]).wait()
            pltpu.make_async_copy(pq_hbm.at[slab.at[j, 1]], rq[b], sg[b]).wait()

        def s_start(j, b):
            base = (c0 + j) * CHUNK
            pltpu.async_copy(rp[b], g_hbm.at[pl.ds(base, CHUNK)], so[b])

        def s_wait(j, b):
            base = (c0 + j) * CHUNK
            pltpu.make_async_copy(rp[b], g_hbm.at[pl.ds(base, CHUNK)], so[b]).wait()

        def vadd(b):
            def rowbody(r, carry):
                for k in range(LATENT // 16):
                    cs = pl.ds(k * 16, 16)
                    rp[b][r, cs] = rp[b][r, cs] + rq[b][r, cs]
                return carry

            lax.fori_loop(0, CHUNK, rowbody, 0)

        g_start(0, 0)
        g_start(1, 1)

        def body(j, carry):
            def step(b):
                g_wait(j, b)
                vadd(b)
                s_start(j, b)
                prev = (b + 2) % 3

                @pl.when(j >= 1)
                def _():
                    s_wait(j - 1, prev)

                @pl.when(j + 2 < wch)
                def _():
                    g_start(j + 2, prev)

            for k in range(3):
                @pl.when(lax.rem(j, 3) == k)
                def _(k=k):
                    step(k)

            return carry

        lax.fori_loop(0, wch, body, 0)
        s_wait(wch - 1, (wch - 1) % 3)

        @pl.when(wid < nrem)
        def _():
            cr = wch * NW + wid
            pltpu.sync_copy(sd_hbm.at[c_lo + cr], slab.at[wch])
            pltpu.async_copy(pq_hbm.at[slab.at[wch, 0]], rp0, sg0)
            pltpu.async_copy(pq_hbm.at[slab.at[wch, 1]], rq0, sg0)
            pltpu.make_async_copy(pq_hbm.at[slab.at[wch, 0]], rp0, sg0).wait()
            pltpu.make_async_copy(pq_hbm.at[slab.at[wch, 1]], rq0, sg0).wait()

            def rowbody(r, carry):
                for k in range(LATENT // 16):
                    cs = pl.ds(k * 16, 16)
                    rp0[r, cs] = rp0[r, cs] + rq0[r, cs]
                return carry

            lax.fori_loop(0, CHUNK, rowbody, 0)
            pltpu.sync_copy(rp0, g_hbm.at[pl.ds(cr * CHUNK, CHUNK)])

    return gather


NCH_H = NCHUNKS // 2
EH = NCH_H * CHUNK
_sc_gather_a = _make_gather(0, NCH_H)
_sc_gather_b = _make_gather(NCH_H, NCH_H)


WCHS = NCH_H // (NW // 2)
NREMS = NCH_H - WCHS * (NW // 2)


@functools.partial(
    pl.kernel,
    out_type=jax.ShapeDtypeStruct((NC, N, LATENT), jnp.float32),
    mesh=_mesh,
    scratch_types=[
        pltpu.VMEM((WCHS + 1, 1, CHUNK), jnp.int32),
        pltpu.VMEM((CHUNK, LATENT), jnp.float32),
        pltpu.VMEM((CHUNK, LATENT), jnp.float32),
        pltpu.VMEM_SHARED((N, LATENT), jnp.float32),
        pltpu.SemaphoreType.DMA,
        pltpu.SemaphoreType.DMA,
        pltpu.SemaphoreType.DMA,
        pltpu.SemaphoreType.DMA,
    ],
)
def _sc_scatter(ma_hbm, mb_hbm, dst2_hbm, zeros_hbm, out_hbm,
                slab, rm0, rm1, acc_sh, sl0, sl1, ss0, ss1):
    cid = lax.axis_index("c")
    sid = lax.axis_index("s")
    wid = sid * NC + cid
    r0 = sid * STRIPE
    rm = (rm0, rm1)
    sl = (sl0, sl1)
    ss = (ss0, ss1)

    @pl.when(sid < NS - 1)
    def _():
        pltpu.sync_copy(zeros_hbm.at[pl.ds(r0, STRIPE)],
                        acc_sh.at[pl.ds(r0, STRIPE)])

    @pl.when(sid == NS - 1)
    def _():
        pltpu.sync_copy(zeros_hbm.at[pl.ds(r0, STRIPE_LAST)],
                        acc_sh.at[pl.ds(r0, STRIPE_LAST)])

    plsc.subcore_barrier()

    def run_half(m_hbm, c_half, wid_local):
        c0 = wid_local * WCHS

        pltpu.sync_copy(dst2_hbm.at[pl.ds(c_half + c0, WCHS)],
                        slab.at[pl.ds(0, WCHS)])

        def l_start(j, b):
            base = (c0 + j) * CHUNK
            pltpu.async_copy(m_hbm.at[pl.ds(base, CHUNK)], rm[b], sl[b])

        def l_wait(j, b):
            base = (c0 + j) * CHUNK
            pltpu.make_async_copy(m_hbm.at[pl.ds(base, CHUNK)], rm[b], sl[b]).wait()

        def sc_start(j, b):
            pltpu.async_copy(rm[b], acc_sh.at[slab.at[j, 0]], ss[b], add=True)

        def sc_wait(j, b):
            pltpu.make_async_copy(rm[b], acc_sh.at[slab.at[j, 0]], ss[b]).wait()

        l_start(0, 0)

        def body(j, carry):
            def step(b):
                l_wait(j, b)
                sc_start(j, b)
                nb = 1 - b

                @pl.when(j >= 1)
                def _():
                    sc_wait(j - 1, nb)

                @pl.when(j + 1 < WCHS)
                def _():
                    l_start(j + 1, nb)

            @pl.when(lax.rem(j, 2) == 0)
            def _():
                step(0)

            @pl.when(lax.rem(j, 2) == 1)
            def _():
                step(1)

            return carry

        lax.fori_loop(0, WCHS, body, 0)
        sc_wait(WCHS - 1, (WCHS - 1) % 2)

        @pl.when(wid_local < NREMS)
        def _():
            cr = WCHS * (NW // 2) + wid_local
            pltpu.sync_copy(m_hbm.at[pl.ds(cr * CHUNK, CHUNK)], rm0)
            pltpu.sync_copy(dst2_hbm.at[c_half + cr], slab.at[WCHS])
            pltpu.sync_copy(rm0, acc_sh.at[slab.at[WCHS, 0]], add=True)

    @pl.when(wid < NW // 2)
    def _():
        run_half(ma_hbm, 0, wid)

    @pl.when(wid >= NW // 2)
    def _():
        run_half(mb_hbm, NCH_H, wid - NW // 2)

    plsc.subcore_barrier()

    @pl.when(sid < NS - 1)
    def _():
        pltpu.sync_copy(acc_sh.at[pl.ds(r0, STRIPE)],
                        out_hbm.at[cid].at[pl.ds(r0, STRIPE)])

    @pl.when(sid == NS - 1)
    def _():
        pltpu.sync_copy(acc_sh.at[pl.ds(r0, STRIPE_LAST)],
                        out_hbm.at[cid].at[pl.ds(r0, STRIPE_LAST)])


def _ln(m, g, b):
    mu = jnp.mean(m, axis=-1, keepdims=True)
    var = jnp.mean((m - mu) ** 2, axis=-1, keepdims=True)
    return g * (m - mu) * lax.rsqrt(var + 1e-5) + b


def _enc_node_body(x_ref, we, be, ws0, wd0, oh, opq):
    h = jnp.dot(x_ref[...], we[...], preferred_element_type=jnp.float32) + be[...]
    oh[...] = h
    opq[0, :, :] = jnp.dot(h, ws0[...], preferred_element_type=jnp.float32)
    opq[1, :, :] = jnp.dot(h, wd0[...], preferred_element_type=jnp.float32)


def _enc_edge_body(a_ref, we, be, oe):
    enc = jnp.dot(a_ref[...], we[...], preferred_element_type=jnp.float32) + be[...]
    oe[...] = enc.astype(jnp.bfloat16)


def _edge_body(e_ref, g_ref, w1e, b1, w2, b2, lng, lnb, om):
    t = jnp.dot(e_ref[...], w1e[...].astype(jnp.bfloat16),
                preferred_element_type=jnp.float32)
    t = t + g_ref[...] + b1[...]
    t = jnp.maximum(t, 0.0)
    m = jnp.dot(t, w2[...], preferred_element_type=jnp.float32) + b2[...]
    m = jnp.maximum(m, 0.0)
    om[...] = _ln(m, lng[...], lnb[...])


def _node_body(h_ref, p0_ref, p1_ref, u1h, u1p, bu1, u2, bu2, lng, lnb,
               wsn, wdn, oh, opq):
    pooled = p0_ref[...] + p1_ref[...]
    t = (jnp.dot(h_ref[...], u1h[...], preferred_element_type=jnp.float32)
         + jnp.dot(pooled, u1p[...], preferred_element_type=jnp.float32)
         + bu1[...])
    t = jnp.maximum(t, 0.0)
    u = jnp.dot(t, u2[...], preferred_element_type=jnp.float32) + bu2[...]
    u = jnp.maximum(u, 0.0)
    hn = h_ref[...] + _ln(u, lng[...], lnb[...])
    oh[...] = hn
    opq[0, :, :] = jnp.dot(hn, wsn[...], preferred_element_type=jnp.float32)
    opq[1, :, :] = jnp.dot(hn, wdn[...], preferred_element_type=jnp.float32)


def _node_final_body(h_ref, p0_ref, p1_ref, u1h, u1p, bu1, u2, bu2, lng, lnb,
                     wdec, bdec, oy):
    pooled = p0_ref[...] + p1_ref[...]
    t = (jnp.dot(h_ref[...], u1h[...], preferred_element_type=jnp.float32)
         + jnp.dot(pooled, u1p[...], preferred_element_type=jnp.float32)
         + bu1[...])
    t = jnp.maximum(t, 0.0)
    u = jnp.dot(t, u2[...], preferred_element_type=jnp.float32) + bu2[...]
    u = jnp.maximum(u, 0.0)
    hn = h_ref[...] + _ln(u, lng[...], lnb[...])
    oy[...] = jnp.dot(hn, wdec[...], preferred_element_type=jnp.float32) + bdec[...]


def _row_spec(bm, width):
    return pl.BlockSpec((bm, width), lambda i: (i, 0))


def _full_spec(shape):
    return pl.BlockSpec(shape, lambda i: tuple(0 for _ in shape))


BN = 2000
BE = 4000


_PQ_SPEC = pl.BlockSpec((2, BN, LATW), lambda i: (0, i, 0))


def _enc_node_call(x, we, be, ws0, wd0):
    return pl.pallas_call(
        _enc_node_body,
        grid=(N // BN,),
        in_specs=[_row_spec(BN, 256), _full_spec((256, LATENT)),
                  _full_spec((1, LATENT)), _full_spec((LATENT, LATENT)),
                  _full_spec((LATENT, LATENT))],
        out_specs=[_row_spec(BN, LATENT), _PQ_SPEC],
        out_shape=[jax.ShapeDtypeStruct((N, LATENT), jnp.float32),
                   jax.ShapeDtypeStruct((2, N, LATW), jnp.float32)],
    )(x, we, be, ws0, wd0)


def _enc_edge_call(attr, we, be):
    return pl.pallas_call(
        _enc_edge_body,
        grid=(E // BE,),
        in_specs=[_row_spec(BE, 16), _full_spec((16, LATENT)),
                  _full_spec((1, LATENT))],
        out_specs=_row_spec(BE, LATENT),
        out_shape=jax.ShapeDtypeStruct((E, LATENT), jnp.bfloat16),
    )(attr, we, be)


def _edge_call(e, g, w1e, b1, w2, b2, lng, lnb, off_blocks):
    w = _full_spec((LATENT, LATENT))
    v = _full_spec((1, LATENT))
    e_spec = pl.BlockSpec((BE, LATENT), lambda i: (i + off_blocks, 0))
    return pl.pallas_call(
        _edge_body,
        grid=(EH // BE,),
        in_specs=[e_spec, _row_spec(BE, LATENT), w, v, w, v, v, v],
        out_specs=_row_spec(BE, LATENT),
        out_shape=jax.ShapeDtypeStruct((EH, LATENT), jnp.float32),
    )(e, g, w1e, b1, w2, b2, lng, lnb)


def _node_call(h, p0, p1, u1h, u1p, bu1, u2, bu2, lng, lnb, wsn, wdn):
    w = _full_spec((LATENT, LATENT))
    v = _full_spec((1, LATENT))
    return pl.pallas_call(
        _node_body,
        grid=(N // BN,),
        in_specs=[_row_spec(BN, LATENT)] * 3 + [w, w, v, w, v, v, v, w, w],
        out_specs=[_row_spec(BN, LATENT), _PQ_SPEC],
        out_shape=[jax.ShapeDtypeStruct((N, LATENT), jnp.float32),
                   jax.ShapeDtypeStruct((2, N, LATW), jnp.float32)],
    )(h, p0, p1, u1h, u1p, bu1, u2, bu2, lng, lnb, wsn, wdn)


def _node_final_call(h, p0, p1, u1h, u1p, bu1, u2, bu2, lng, lnb, wdec, bdec):
    w = _full_spec((LATENT, LATENT))
    v = _full_spec((1, LATENT))
    return pl.pallas_call(
        _node_final_body,
        grid=(N // BN,),
        in_specs=[_row_spec(BN, LATENT)] * 3 + [w, w, v, w, v, v, v, w, v],
        out_specs=_row_spec(BN, LATENT),
        out_shape=jax.ShapeDtypeStruct((N, LATENT), jnp.float32),
    )(h, p0, p1, u1h, u1p, bu1, u2, bu2, lng, lnb, wdec, bdec)


def kernel(x, edge_index, edge_attr, params):
    src = edge_index[0]
    dst = edge_index[1]
    dst2 = dst.reshape(NCHUNKS, 1, CHUNK)
    sd = jnp.stack([src, dst + N], axis=0).reshape(2, NCHUNKS, CHUNK)
    sd = sd.transpose(1, 0, 2)
    layers = params["layers"]

    def row(v):
        return v.reshape(1, -1)

    w1 = [lp["msg1"]["w"] for lp in layers]
    w1e = [w[0:LATENT] for w in w1]
    w1s = [w[LATENT:2 * LATENT] for w in w1]
    w1d = [w[2 * LATENT:3 * LATENT] for w in w1]
    u1 = [lp["upd1"]["w"] for lp in layers]
    u1h = [w[0:LATENT] for w in u1]
    u1p = [w[LATENT:2 * LATENT] for w in u1]

    wdec = jnp.zeros((LATENT, LATENT), jnp.float32).at[:, :3].set(params["dec"]["w"])
    bdec = jnp.zeros((1, LATENT), jnp.float32).at[:, :3].set(params["dec"]["b"])
    zeros_n = jnp.zeros((N, LATENT), jnp.float32)

    h, pq = _enc_node_call(x, params["enc_node"]["w"], row(params["enc_node"]["b"]),
                           w1s[0], w1d[0])
    e = _enc_edge_call(edge_attr, params["enc_edge"]["w"], row(params["enc_edge"]["b"]))

    for l in range(NUM_MP):
        lp = layers[l]
        pqf = pq.reshape(2 * N, LATENT)
        ga = _sc_gather_a(pqf, sd)
        gb = _sc_gather_b(pqf, sd)
        ew = (w1e[l], row(lp["msg1"]["b"]), lp["msg2"]["w"], row(lp["msg2"]["b"]),
              row(lp["msg_ln_g"]), row(lp["msg_ln_b"]))
        ma = _edge_call(e, ga, *ew, 0)
        mb = _edge_call(e, gb, *ew, EH // BE)
        pooled2 = _sc_scatter(ma, mb, dst2, zeros_n)
        args = (h, pooled2[0], pooled2[1], u1h[l], u1p[l], row(lp["upd1"]["b"]),
                lp["upd2"]["w"], row(lp["upd2"]["b"]),
                row(lp["upd_ln_g"]), row(lp["upd_ln_b"]))
        if l < NUM_MP - 1:
            h, pq = _node_call(*args, w1s[l + 1], w1d[l + 1])
        else:
            y = _node_final_call(*args, wdec, bdec)

    return y[:, :3]
